# baseline (speedup 1.0000x reference)
import functools

import jax
import jax.numpy as jnp
from jax import lax
from jax.experimental import pallas as pl
from jax.experimental.pallas import tpu as pltpu
from jax.experimental.pallas import tpu_sc as plsc

D = 32
NW = 32
CHUNK = 128
GROUP = 8
NBI = 4
NBR = 2


def _make_kernel(n_ch):
    n_groups = n_ch // GROUP
    assert n_groups % 4 == 0 and n_groups >= 8
    mesh = plsc.VectorSubcoreMesh(core_axis_name="c", subcore_axis_name="s")

    @functools.partial(
        pl.kernel,
        mesh=mesh,
        out_type=jax.ShapeDtypeStruct((NW, n_ch, CHUNK, D), jnp.float32),
        scratch_types=[
            pltpu.VMEM((NBI, GROUP, CHUNK), jnp.int32),
            pltpu.VMEM((NBR, GROUP, CHUNK, D), jnp.float32),
        ]
        + [pltpu.SemaphoreType.DMA] * (NBI + NBR + NBR),
        compiler_params=pltpu.CompilerParams(use_tc_tiling_on_sc=False),
    )
    def k(idx_hbm, table_hbm, out_hbm, idx_v, rows_v, *sems):
        sem_i = sems[:NBI]
        sem_g = sems[NBI:NBI + NBR]
        sem_o = sems[NBI + NBR:]
        wid = lax.axis_index("s") * 2 + lax.axis_index("c")

        def idx_desc(g, s):
            return pltpu.make_async_copy(
                idx_hbm.at[wid, pl.ds(g * GROUP, GROUP)], idx_v.at[s], sem_i[s])

        def gather_descs(s2, s4):
            return [
                pltpu.make_async_copy(
                    table_hbm.at[idx_v.at[s4, j]], rows_v.at[s2, j], sem_g[s2])
                for j in range(GROUP)
            ]

        def out_desc(g, s):
            return pltpu.make_async_copy(
                rows_v.at[s], out_hbm.at[wid, pl.ds(g * GROUP, GROUP)], sem_o[s])

        def fire_gathers(s2, s4):
            for d in gather_descs(s2, s4):
                d.start()

        def wait_gathers(s2, s4):
            for d in gather_descs(s2, s4):
                d.wait()

        for t in range(NBI):
            idx_desc(t, t).start()
        idx_desc(0, 0).wait()
        fire_gathers(0, 0)
        idx_desc(1, 1).wait()
        fire_gathers(1, 1)
        wait_gathers(0, 0)
        out_desc(0, 0).start()
        idx_desc(4, 0).start()
        idx_desc(2, 2).wait()
        out_desc(0, 0).wait()
        fire_gathers(0, 2)
        wait_gathers(1, 1)
        out_desc(1, 1).start()
        idx_desc(5, 1).start()
        idx_desc(3, 3).wait()
        out_desc(1, 1).wait()
        fire_gathers(1, 3)
        wait_gathers(0, 2)
        out_desc(2, 0).start()
        idx_desc(6, 2).start()

        def body(q, carry):
            g0 = q * 4
            for r in range(4):
                g = g0 + r
                s2, s4 = r % 2, r
                idx_desc(g, s4).wait()
                out_desc(g - 2, s2).wait()
                fire_gathers(s2, s4)
                wait_gathers((r - 1) % 2, (r - 1) % 4)
                out_desc(g - 1, (r - 1) % 2).start()
                idx_desc(jnp.minimum(g + 3, n_groups - 1), (r + 3) % 4).start()
            return carry

        lax.fori_loop(1, n_groups // 4, body, 0)

        last = n_groups - 1
        wait_gathers(last % 2, last % 4)
        out_desc(last, last % 2).start()
        out_desc(last - 1, (last - 1) % 2).wait()
        out_desc(last, last % 2).wait()
        for s in range(3):
            idx_desc(last, s).wait()

    return k


def kernel(widths, table):
    B, H = widths.shape
    total = B * H
    n_ch = total // (NW * CHUNK)
    idx = widths.reshape(NW, n_ch, CHUNK)
    out = _make_kernel(n_ch)(idx, table)
    return out.reshape(B, H, D)


# speedup vs baseline: 1.0430x; 1.0430x over previous
import functools

import jax
import jax.numpy as jnp
from jax import lax
from jax.experimental import pallas as pl
from jax.experimental.pallas import tpu as pltpu
from jax.experimental.pallas import tpu_sc as plsc

D = 32
NW = 32
CB = 1024
UNROLL = 16


def _make_kernel(n_idx, n_rows):
    n_steps = n_idx // CB
    assert n_steps % 2 == 0 and n_steps >= 4
    mesh = plsc.VectorSubcoreMesh(core_axis_name="c", subcore_axis_name="s")

    @functools.partial(
        pl.kernel,
        mesh=mesh,
        out_type=jax.ShapeDtypeStruct((NW, n_idx * D), jnp.float32),
        scratch_types=[
            pltpu.VMEM((n_rows * D,), jnp.float32),
            pltpu.VMEM((2, CB), jnp.int32),
            pltpu.VMEM((2, CB * D), jnp.float32),
        ]
        + [pltpu.SemaphoreType.DMA] * 5,
        compiler_params=pltpu.CompilerParams(use_tc_tiling_on_sc=False),
    )
    def k(idx_hbm, table_hbm, out_hbm, table_v, idx_v, rows_v, *sems):
        sem_t = sems[0]
        sem_i = sems[1:3]
        sem_o = sems[3:5]
        wid = lax.axis_index("s") * 2 + lax.axis_index("c")

        def idx_desc(s, b):
            return pltpu.make_async_copy(
                idx_hbm.at[wid, pl.ds(s * CB, CB)], idx_v.at[b], sem_i[b])

        def out_desc(s, b):
            return pltpu.make_async_copy(
                rows_v.at[b], out_hbm.at[wid, pl.ds(s * CB * D, CB * D)],
                sem_o[b])

        tab_desc = pltpu.make_async_copy(table_hbm, table_v, sem_t)
        tab_desc.start()
        idx_desc(0, 0).start()
        idx_desc(1, 1).start()
        tab_desc.wait()

        def compute(b):
            def cbody(t, carry):
                o = t * UNROLL
                vbase = idx_v[b, pl.ds(o, UNROLL)] * D
                for i in range(UNROLL):
                    base = vbase[i]
                    ob = (o + i) * D
                    rows_v[b, pl.ds(ob, 16)] = table_v[pl.ds(base, 16)]
                    rows_v[b, pl.ds(ob + 16, 16)] = table_v[pl.ds(base + 16, 16)]
                return carry

            lax.fori_loop(0, CB // UNROLL, cbody, 0)

        def step(s, b, first):
            idx_desc(s, b).wait()
            if not first:
                out_desc(s - 2, b).wait()
            compute(b)
            out_desc(s, b).start()
            idx_desc(jnp.minimum(s + 2, n_steps - 1), b).start()

        step(0, 0, True)
        step(1, 1, True)

        def body(p, carry):
            step(p * 2, 0, False)
            step(p * 2 + 1, 1, False)
            return carry

        lax.fori_loop(1, n_steps // 2, body, 0)

        out_desc(n_steps - 2, 0).wait()
        out_desc(n_steps - 1, 1).wait()
        idx_desc(n_steps - 1, 0).wait()
        idx_desc(n_steps - 1, 1).wait()

    return k


def kernel(widths, table):
    B, H = widths.shape
    n_rows = table.shape[0]
    n_idx = B * H // NW
    idx = widths.reshape(NW, n_idx)
    out = _make_kernel(n_idx, n_rows)(idx, table.reshape(-1))
    return out.reshape(B, H, D)
